# baseline (speedup 1.0000x reference)
import functools

import jax
import jax.numpy as jnp
from jax.experimental import pallas as pl
from jax.experimental.pallas import tpu as pltpu

N_TOK = 4096
D = 32
K = 8192
BN = 512
BK = 2048


def _argmin_body(feat_ref, cb_ref, idx_ref):
    f = feat_ref[...]
    fn = f / (jnp.sqrt(jnp.sum(f * f, axis=1, keepdims=True)) + 1e-12)

    def step(k, carry):
        rmin, ridx = carry
        c = cb_ref[pl.ds(k * BK, BK), :]
        cn = c / (jnp.sqrt(jnp.sum(c * c, axis=1, keepdims=True)) + 1e-12)
        d = 1.0 - jax.lax.dot_general(
            fn, cn, dimension_numbers=(((1,), (1,)), ((), ())),
            preferred_element_type=jnp.float32)
        dmin = jnp.min(d, axis=1, keepdims=True)
        kio = jax.lax.broadcasted_iota(jnp.int32, d.shape, 1) + k * BK
        didx = jnp.min(
            jnp.where(d == dmin, kio, jnp.int32(2**31 - 1)),
            axis=1, keepdims=True)
        better = dmin < rmin
        return jnp.where(better, dmin, rmin), jnp.where(better, didx, ridx)

    init = (jnp.full((BN, 1), jnp.inf, jnp.float32),
            jnp.zeros((BN, 1), jnp.int32))
    rmin, ridx = jax.lax.fori_loop(0, K // BK, step, init)
    idx_ref[0, 0, :] = ridx[:, 0]


def _nearest_idx(z, codebook):
    feat = z.reshape(N_TOK, D)
    nb = N_TOK // BN
    idx3 = pl.pallas_call(
        _argmin_body,
        grid=(nb,),
        in_specs=[
            pl.BlockSpec((BN, D), lambda n: (n, 0)),
            pl.BlockSpec((K, D), lambda n: (0, 0)),
        ],
        out_specs=pl.BlockSpec((1, 1, BN), lambda n: (n, 0, 0)),
        out_shape=jax.ShapeDtypeStruct((nb, 1, BN), jnp.int32),
    )(feat, codebook)
    return idx3.reshape(N_TOK)


def kernel(z, codebook, cluster_labels):
    bs = z.shape[0]
    idx = _nearest_idx(z, codebook)
    token_labels = jnp.take(cluster_labels, idx, axis=0)
    img = token_labels.reshape(bs, 1, 16, 16).astype(jnp.float32)
    out = jnp.repeat(jnp.repeat(img, 14, axis=2), 14, axis=3)
    return out


# speedup vs baseline: 1.1120x; 1.1120x over previous
import functools

import jax
import jax.numpy as jnp
from jax.experimental import pallas as pl
from jax.experimental.pallas import tpu as pltpu

N_TOK = 4096
D = 32
K = 8192
BN = 512


def _norm_body(feat_ref, cb_ref, fn_ref, cn_ref):
    f = feat_ref[...]
    fn_ref[...] = f / (jnp.sqrt(jnp.sum(f * f, axis=1, keepdims=True)) + 1e-12)
    c = cb_ref[...]
    cn_ref[...] = c / (jnp.sqrt(jnp.sum(c * c, axis=1, keepdims=True)) + 1e-12)


def _argmin_body(fn_ref, cn_ref, ki_ref, idx_ref):
    s = jax.lax.dot_general(
        fn_ref[...], cn_ref[...],
        dimension_numbers=(((1,), (1,)), ((), ())),
        preferred_element_type=jnp.float32)
    d = 1.0 - s
    dmin = jnp.min(d, axis=1, keepdims=True)
    midx = jnp.min(
        jnp.where(d == dmin, ki_ref[...], jnp.int32(2**31 - 1)),
        axis=1, keepdims=True)
    idx_ref[...] = midx


def _nearest_idx(z, codebook):
    feat = z.reshape(N_TOK, D)
    fn, cn = pl.pallas_call(
        _norm_body,
        grid=(1,),
        in_specs=[
            pl.BlockSpec((N_TOK, D), lambda i: (0, 0)),
            pl.BlockSpec((K, D), lambda i: (0, 0)),
        ],
        out_specs=[
            pl.BlockSpec((N_TOK, D), lambda i: (0, 0)),
            pl.BlockSpec((K, D), lambda i: (0, 0)),
        ],
        out_shape=[
            jax.ShapeDtypeStruct((N_TOK, D), jnp.float32),
            jax.ShapeDtypeStruct((K, D), jnp.float32),
        ],
    )(feat, codebook)
    ki = jax.lax.broadcasted_iota(jnp.int32, (1, K), 1)
    nb = N_TOK // BN
    idx2 = pl.pallas_call(
        _argmin_body,
        grid=(nb,),
        in_specs=[
            pl.BlockSpec((BN, D), lambda n: (n, 0)),
            pl.BlockSpec((K, D), lambda n: (0, 0)),
            pl.BlockSpec((1, K), lambda n: (0, 0)),
        ],
        out_specs=pl.BlockSpec((BN, 1), lambda n: (n, 0)),
        out_shape=jax.ShapeDtypeStruct((N_TOK, 1), jnp.int32),
    )(fn, cn, ki)
    return idx2.reshape(N_TOK)


def kernel(z, codebook, cluster_labels):
    bs = z.shape[0]
    idx = _nearest_idx(z, codebook)
    token_labels = jnp.take(cluster_labels, idx, axis=0)
    img = token_labels.reshape(bs, 1, 16, 16).astype(jnp.float32)
    out = jnp.repeat(jnp.repeat(img, 14, axis=2), 14, axis=3)
    return out
